# baseline (speedup 1.0000x reference)
import functools

import jax
import jax.numpy as jnp
import numpy as np
from jax.experimental import pallas as pl

_EPS = 1e-3
_B, _N, _F, _T, _H = 2, 384, 128, 8, 64
_INV_DEG = 1.0 / (_N - 1)
_RSQ = 1.0 / np.sqrt(1.0 + _EPS)

_LAYER_DIMS = (
    ("tm1", _T, _H), ("tm2", _H, _H),
    ("pre1", _F, _H), ("pre2", _H, _H),
    ("c1p1", 2 * _H, _H), ("c1p2", _H, _H),
    ("c1u1", 3 * _H, _H), ("c1u2", _H, _H),
    ("c2p1", 2 * _H, _H), ("c2p2", _H, _H),
    ("c2u1", 3 * _H, _H), ("c2u2", _H, _H),
    ("post1", _H, _H), ("post2", _H, _H),
)
_OFF = {}
_r = 0
for _name, _d, _u in _LAYER_DIMS:
    _gr = -(-_d // 128)
    _OFF[_name] = {"ga": _r, "be": _r + _gr, "b": _r + 2 * _gr,
                   "W": _r + 2 * _gr + 1, "d": _d, "u": _u}
    _r += 2 * _gr + 1 + _d
_OFF["lgW"] = _r
_OFF["lgb"] = _r + _H
_ROWS = _r + _H + 1


def _rows_per_batch(v):
    d = v.shape[1]
    return jnp.concatenate(
        [jnp.broadcast_to(v[i:i + 1], (_N, d)) for i in range(_B)], axis=0)


def _mean_of_others(g):
    d = g.shape[1]
    s = jnp.concatenate(
        [jnp.broadcast_to(jnp.sum(g[i * _N:(i + 1) * _N], axis=0,
                                  keepdims=True), (_N, d))
         for i in range(_B)], axis=0)
    return (s - g) * _INV_DEG


def _gnn_body(time_ref, p_ref, w_ref, out_ref):
    dot = functools.partial(jnp.dot, precision=jax.lax.Precision.DEFAULT,
                            preferred_element_type=jnp.float32)
    gelu = jax.nn.gelu

    def vec(off, c):
        return w_ref[off:off + 1, 0:c]

    def chunk(base, k):
        return w_ref[base + k // 2:base + k // 2 + 1,
                     64 * (k % 2):64 * (k % 2) + 64]

    def bn_chunk(v, name, k):
        o = _OFF[name]
        return v * (chunk(o["ga"], k) * _RSQ) + chunk(o["be"], k)

    def w_chunk(name, k):
        o = _OFF[name]
        return w_ref[o["W"] + _H * k:o["W"] + _H * (k + 1), 0:o["u"]]

    def layer(v, name):
        o = _OFF[name]
        d, u = o["d"], o["u"]
        vn = v * (vec(o["ga"], d) * _RSQ) + vec(o["be"], d)
        return gelu(dot(vn, w_ref[o["W"]:o["W"] + d, 0:u]) + vec(o["b"], u))

    t = layer(layer(time_ref[...], "tm1"), "tm2")

    x = layer(layer(p_ref[...], "pre1"), "pre2")

    for pn1, pn2, un1, un2 in (("c1p1", "c1p2", "c1u1", "c1u2"),
                               ("c2p1", "c2p2", "c2u1", "c2u2")):
        pb = vec(_OFF[pn1]["b"], _H)
        tb = dot(bn_chunk(t, pn1, 1), w_chunk(pn1, 1)) + pb
        g = gelu(dot(bn_chunk(x, pn1, 0), w_chunk(pn1, 0))
                 + _rows_per_batch(tb))
        g = layer(g, pn2)
        agg = _mean_of_others(g)
        ub = vec(_OFF[un1]["b"], _H)
        utb = dot(bn_chunk(t, un1, 2), w_chunk(un1, 2)) + ub
        u = gelu(dot(bn_chunk(x, un1, 0), w_chunk(un1, 0))
                 + dot(bn_chunk(agg, un1, 1), w_chunk(un1, 1))
                 + _rows_per_batch(utb))
        u = layer(u, un2)
        x = x + u

    x = layer(layer(x, "post1"), "post2")
    lw = w_ref[_OFF["lgW"]:_OFF["lgW"] + _H, 0:_F]
    out_ref[...] = dot(x, lw) + vec(_OFF["lgb"], _F)


def _pad_row(a):
    a = a[None, :] if a.ndim == 1 else a
    pad = -a.shape[1] % 128
    a = jnp.pad(a, ((0, 0), (0, pad))) if pad else a
    return a.reshape(-1, 128)


def kernel(p, time, params):
    order = ("time_mlp", "pre", "c1_prep", "c1_upd", "c2_prep", "c2_upd",
             "post")
    pieces = []
    for key in order:
        for lay in params[key]:
            pieces += [_pad_row(lay["gamma"]), _pad_row(lay["beta"]),
                       _pad_row(lay["b"]), _pad_row(lay["W"])]
    pieces += [params["logits_W"], _pad_row(params["logits_b"])]
    packed = jnp.concatenate(pieces, axis=0)

    out = pl.pallas_call(
        _gnn_body,
        out_shape=jax.ShapeDtypeStruct((_B * _N, _F), jnp.float32),
    )(time, p.reshape(_B * _N, _F), packed)
    return out.reshape(_B, _N, _F)


# speedup vs baseline: 3.5439x; 3.5439x over previous
import functools

import jax
import jax.numpy as jnp
import numpy as np
from jax.experimental import pallas as pl

_EPS = 1e-3
_B, _N, _F, _T, _H = 2, 384, 128, 8, 64
_INV_DEG = 1.0 / (_N - 1)
_RSQ = 1.0 / np.sqrt(1.0 + _EPS)

_LAYERS = {
    "tm1": ("time_mlp", 0, _T), "tm2": ("time_mlp", 1, _H),
    "pre1": ("pre", 0, _F), "pre2": ("pre", 1, _H),
    "c1p1": ("c1_prep", 0, 2 * _H), "c1p2": ("c1_prep", 1, _H),
    "c1u1": ("c1_upd", 0, 3 * _H), "c1u2": ("c1_upd", 1, _H),
    "c2p1": ("c2_prep", 0, 2 * _H), "c2p2": ("c2_prep", 1, _H),
    "c2u1": ("c2_upd", 0, 3 * _H), "c2u2": ("c2_upd", 1, _H),
    "post1": ("post", 0, _H), "post2": ("post", 1, _H),
}
_LAYER_ORDER = ("tm1", "tm2", "pre1", "pre2", "c1p1", "c1p2", "c1u1", "c1u2",
                "c2p1", "c2p2", "c2u1", "c2u2", "post1", "post2")

_WOFF = {}
_r = 0
for _nm in _LAYER_ORDER:
    _WOFF[_nm] = _r
    _r += _LAYERS[_nm][2]

_VPIECES = ["lgW", "lgb"]
for _nm in ("pre1", "c1p1", "c2p1", "c1u1", "c2u1",
            "tm2", "pre2", "c1p2", "c1u2", "c2p2", "c2u2", "post1", "post2"):
    _VPIECES += [_nm + ":ga", _nm + ":be"]
_VPIECES += [_nm + ":b" for _nm in _LAYER_ORDER]
_VPIECES += ["tm1:ga", "tm1:be"]

_VSIZE = {"lgW": _H * _F, "lgb": _F}
for _nm in _LAYER_ORDER:
    _d = _LAYERS[_nm][2]
    _VSIZE[_nm + ":ga"] = _d
    _VSIZE[_nm + ":be"] = _d
    _VSIZE[_nm + ":b"] = _H

_VOFF = {}
_v = 0
for _pc in _VPIECES:
    _VOFF[_pc] = _v
    _v += _VSIZE[_pc]
_VPAD = -_v % 128
_VROWS = (_v + _VPAD) // 128


def _rows_per_batch(v):
    d = v.shape[1]
    return jnp.concatenate(
        [jnp.broadcast_to(v[i:i + 1], (_N, d)) for i in range(_B)], axis=0)


def _mean_of_others(g):
    d = g.shape[1]
    s = jnp.concatenate(
        [jnp.broadcast_to(jnp.sum(g[i * _N:(i + 1) * _N], axis=0,
                                  keepdims=True), (_N, d))
         for i in range(_B)], axis=0)
    return (s - g) * _INV_DEG


def _gnn_body(time_ref, p_ref, w64_ref, vec_ref, out_ref):
    dot = functools.partial(jnp.dot, precision=jax.lax.Precision.DEFAULT,
                            preferred_element_type=jnp.float32)
    gelu = jax.nn.gelu

    def vrow(pc, c):
        o = _VOFF[pc]
        return vec_ref[o // 128:o // 128 + 1, o % 128:o % 128 + c]

    def vchunk(pc, k):
        o = _VOFF[pc] + 64 * k
        return vec_ref[o // 128:o // 128 + 1, o % 128:o % 128 + 64]

    def weight(nm, lo=0, hi=None):
        o = _WOFF[nm]
        hi = _LAYERS[nm][2] if hi is None else hi
        return w64_ref[o + lo:o + hi, 0:64]

    def bn(v, nm):
        d = v.shape[1]
        return v * (vrow(nm + ":ga", d) * _RSQ) + vrow(nm + ":be", d)

    def bn_chunk(v, nm, k):
        return v * (vchunk(nm + ":ga", k) * _RSQ) + vchunk(nm + ":be", k)

    def layer(v, nm):
        return gelu(dot(bn(v, nm), weight(nm)) + vrow(nm + ":b", _H))

    t = layer(layer(time_ref[...], "tm1"), "tm2")

    x = layer(layer(p_ref[...], "pre1"), "pre2")

    for pn1, pn2, un1, un2 in (("c1p1", "c1p2", "c1u1", "c1u2"),
                               ("c2p1", "c2p2", "c2u1", "c2u2")):
        tb = dot(bn_chunk(t, pn1, 1), weight(pn1, _H)) + vrow(pn1 + ":b", _H)
        g = gelu(dot(bn_chunk(x, pn1, 0), weight(pn1, 0, _H))
                 + _rows_per_batch(tb))
        g = layer(g, pn2)
        agg = _mean_of_others(g)
        utb = (dot(bn_chunk(t, un1, 2), weight(un1, 2 * _H))
               + vrow(un1 + ":b", _H))
        u = gelu(dot(bn_chunk(x, un1, 0), weight(un1, 0, _H))
                 + dot(bn_chunk(agg, un1, 1), weight(un1, _H, 2 * _H))
                 + _rows_per_batch(utb))
        u = layer(u, un2)
        x = x + u

    x = layer(layer(x, "post1"), "post2")
    lw = vec_ref[_VOFF["lgW"] // 128:_VOFF["lgW"] // 128 + _H, 0:_F]
    out_ref[...] = dot(x, lw) + vrow("lgb", _F)


def kernel(p, time, params):
    def leaf(pc):
        if pc == "lgW":
            return params["logits_W"]
        if pc == "lgb":
            return params["logits_b"]
        nm, field = pc.split(":")
        key, idx, _ = _LAYERS[nm]
        return params[key][idx][{"ga": "gamma", "be": "beta", "b": "b"}[field]]

    w64 = jnp.concatenate(
        [params[_LAYERS[nm][0]][_LAYERS[nm][1]]["W"] for nm in _LAYER_ORDER],
        axis=0)
    vec = jnp.concatenate(
        [leaf(pc).ravel() for pc in _VPIECES]
        + [jnp.zeros((_VPAD,), jnp.float32)]).reshape(_VROWS, 128)

    out = pl.pallas_call(
        _gnn_body,
        out_shape=jax.ShapeDtypeStruct((_B * _N, _F), jnp.float32),
    )(time, p.reshape(_B * _N, _F), w64, vec)
    return out.reshape(_B, _N, _F)


# speedup vs baseline: 7.0061x; 1.9769x over previous
import functools

import jax
import jax.numpy as jnp
import numpy as np
from jax.experimental import pallas as pl

_EPS = 1e-3
_B, _N, _F, _T, _H = 2, 384, 128, 8, 64
_INV_DEG = 1.0 / (_N - 1)
_RSQ = 1.0 / np.sqrt(1.0 + _EPS)


def _gnn_body(time_ref, p_ref, *refs):
    out_ref = refs[-1]
    (tm1, tm2, pre1, pre2, c1p1, c1p2, c1u1, c1u2,
     c2p1, c2p2, c2u1, c2u2, post1, post2) = (
        [refs[4 * i:4 * i + 4] for i in range(14)])
    lg_w_ref, lg_b_ref = refs[56], refs[57]

    dot = functools.partial(jnp.dot, precision=jax.lax.Precision.DEFAULT,
                            preferred_element_type=jnp.float32)
    gelu = jax.nn.gelu

    def bdiag(w):
        d, u = w.shape
        z = jnp.zeros((d, u), jnp.float32)
        return jnp.concatenate(
            [jnp.concatenate([w, z], axis=1),
             jnp.concatenate([z, w], axis=1)], axis=0)

    def dup(v):
        return jnp.concatenate([v, v], axis=1)

    def pair(v):
        return jnp.concatenate([v[0:1], v[1:2]], axis=1)

    def bn_bl(x, lp, lo, hi):
        ga_ref, be_ref = lp[0], lp[1]
        return (x * dup(ga_ref[:, lo:hi] * _RSQ) + dup(be_ref[:, lo:hi]))

    def bn_t(v, lp, lo, hi):
        return v * (lp[0][:, lo:hi] * _RSQ) + lp[1][:, lo:hi]

    def layer_bl(x, lp):
        d = lp[2].shape[0]
        return gelu(dot(bn_bl(x, lp, 0, d), bdiag(lp[2][...]))
                    + dup(lp[3][...]))

    t = gelu(dot(bn_t(time_ref[...], tm1, 0, _T), tm1[2][...]) + tm1[3][...])
    t = gelu(dot(bn_t(t, tm2, 0, _H), tm2[2][...]) + tm2[3][...])

    x = jnp.concatenate([p_ref[0], p_ref[1]], axis=1)
    x = layer_bl(x, pre1)
    x = layer_bl(x, pre2)

    for pl1, pl2, ul1, ul2 in ((c1p1, c1p2, c1u1, c1u2),
                               (c2p1, c2p2, c2u1, c2u2)):
        tb = dot(bn_t(t, pl1, _H, 2 * _H), pl1[2][_H:, :]) + pl1[3][...]
        g = gelu(dot(bn_bl(x, pl1, 0, _H), bdiag(pl1[2][:_H, :])) + pair(tb))
        g = layer_bl(g, pl2)
        s = jnp.sum(g, axis=0, keepdims=True)
        agg = (s - g) * _INV_DEG
        utb = (dot(bn_t(t, ul1, 2 * _H, 3 * _H), ul1[2][2 * _H:, :])
               + ul1[3][...])
        u = gelu(dot(bn_bl(x, ul1, 0, _H), bdiag(ul1[2][:_H, :]))
                 + dot(bn_bl(agg, ul1, _H, 2 * _H),
                       bdiag(ul1[2][_H:2 * _H, :]))
                 + pair(utb))
        u = layer_bl(u, ul2)
        x = x + u

    x = layer_bl(x, post1)
    x = layer_bl(x, post2)
    o = dot(x, bdiag(lg_w_ref[...])) + dup(lg_b_ref[...])
    out_ref[0] = o[:, 0:_F]
    out_ref[1] = o[:, _F:2 * _F]


def kernel(p, time, params):
    weights = []
    for key in ("time_mlp", "pre", "c1_prep", "c1_upd", "c2_prep", "c2_upd",
                "post"):
        for lay in params[key]:
            weights += [lay["gamma"][None, :], lay["beta"][None, :],
                        lay["W"], lay["b"][None, :]]
    weights.append(params["logits_W"])
    weights.append(params["logits_b"][None, :])

    out = pl.pallas_call(
        _gnn_body,
        out_shape=jax.ShapeDtypeStruct((_B, _N, _F), jnp.float32),
    )(time, p, *weights)
    return out
